# baseline (speedup 1.0000x reference)
import functools

import jax
import jax.numpy as jnp
from jax import lax
from jax.experimental import pallas as pl
from jax.experimental.pallas import tpu as pltpu
from jax.experimental.pallas import tpu_sc as plsc

H, W = 512, 512
HW = H * W
B = 8
N = 131072
NREG = 8
R = HW // NREG
NTASK = B * NREG
CHUNK = 8192
L = 16
SUBC = 2048
ZSLOT = 3 * B * N

_mesh = plsc.VectorSubcoreMesh(core_axis_name="c", subcore_axis_name="s")


@functools.partial(
    pl.kernel,
    mesh=_mesh,
    out_type=[
        jax.ShapeDtypeStruct((B * NREG * R,), jnp.float32),
        jax.ShapeDtypeStruct((B * 3 * NREG * R,), jnp.float32),
        jax.ShapeDtypeStruct((B * NREG * R,), jnp.int32),
    ],
    scratch_types=[
        pltpu.VMEM((CHUNK,), jnp.float32),
        pltpu.VMEM((CHUNK,), jnp.float32),
        pltpu.VMEM((CHUNK,), jnp.float32),
        pltpu.VMEM((R,), jnp.float32),
        pltpu.VMEM((R,), jnp.int32),
        pltpu.VMEM((2048,), jnp.int32),
        pltpu.VMEM((3 * SUBC,), jnp.int32),
        pltpu.VMEM((6 * SUBC,), jnp.float32),
        pltpu.SemaphoreType.DMA,
        pltpu.SemaphoreType.DMA,
        pltpu.SemaphoreType.DMA,
        pltpu.SemaphoreType.DMA,
    ],
    compiler_params=pltpu.CompilerParams(needs_layout_passes=False),
)
def _sc_project(points_hbm, colors_hbm, depth_hbm, img_hbm, idx_hbm,
                xb, yb, zb, depth_v, id_v, tmp_v, idx3, cstage,
                sem_pt, sem_cg, sem_im, sem_out):
    wid = lax.axis_index("s") * 2 + lax.axis_index("c")
    lane = lax.iota(jnp.int32, L)
    inf16 = jnp.full((L,), jnp.inf, dtype=jnp.float32)
    n16 = jnp.full((L,), N, dtype=jnp.int32)
    nc = N // CHUNK

    for k in range(NTASK // 32):
        t = wid + 32 * k
        b = t >> 3
        q = t & 7
        pbase = b * 4 * N
        cbase = b * 3 * N

        def init_body(i, _):
            depth_v[pl.ds(i * L, L)] = inf16
            id_v[pl.ds(i * L, L)] = n16
            return 0
        lax.fori_loop(0, R // L, init_body, 0)

        def chunk_body(c, _):
            off = c * CHUNK
            pltpu.sync_copy(points_hbm.at[pl.ds(pbase + off, CHUNK)], xb)
            pltpu.sync_copy(points_hbm.at[pl.ds(pbase + N + off, CHUNK)], yb)
            pltpu.sync_copy(points_hbm.at[pl.ds(pbase + 2 * N + off, CHUNK)],
                            zb)

            def process(jbase, lex, dirty):
                s = pl.ds(jbase, L)
                x = xb[s]
                y = yb[s]
                z = zb[s]
                u = jnp.minimum((x * jnp.float32(W)).astype(jnp.int32), W - 1)
                v = jnp.minimum((y * jnp.float32(H)).astype(jnp.int32), H - 1)
                pix = (v << 9) | u
                in_reg = (pix >> 15) == q
                local = pix & (R - 1)
                ids = (off + jbase) + lane
                d0 = plsc.load_gather(depth_v, [local])
                if lex:
                    i0 = plsc.load_gather(id_v, [local])
                    want = in_reg & ((z < d0) | ((z == d0) & (ids < i0)))
                else:
                    want = in_reg & (z < d0)
                slot = local & 2047
                plsc.store_scatter(tmp_v, [slot], lane, mask=want)
                winner = plsc.load_gather(tmp_v, [slot])
                lead = want & (winner == lane)
                plsc.store_scatter(depth_v, [local], z, mask=lead)
                plsc.store_scatter(id_v, [local], ids, mask=lead)
                return dirty | (want ^ lead)

            def make_pass(lex):
                def vec_body(j, dirty):
                    dirty = process(j * 2 * L, lex, dirty)
                    dirty = process(j * 2 * L + L, lex, dirty)
                    return dirty
                return lax.fori_loop(0, CHUNK // (2 * L), vec_body,
                                     jnp.zeros((L,), dtype=jnp.bool_))

            dirty = make_pass(lex=False)
            lax.while_loop(jnp.any, lambda d: make_pass(lex=True), dirty)
            return 0
        lax.fori_loop(0, nc, chunk_body, 0)

        def fin_body(i, _):
            s = pl.ds(i * L, L)
            idv = id_v[s]
            dv = depth_v[s]
            valid = idv < N
            id_v[s] = jnp.where(valid, idv, -1)
            depth_v[s] = jnp.where(valid, dv, 0.0)
            return 0
        lax.fori_loop(0, R // L, fin_body, 0)

        obase = (b * NREG + q) * R
        cp_d = pltpu.async_copy(depth_v, depth_hbm.at[pl.ds(obase, R)],
                                sem_out)
        cp_i = pltpu.async_copy(id_v, idx_hbm.at[pl.ds(obase, R)], sem_out)

        ibase0 = (b * 3 * NREG + q) * R

        def sub_body(sc_i, _):
            sbase = sc_i * SUBC
            cpar = sc_i & 1

            def bld(i, _):
                idv = id_v[pl.ds(sbase + i * L, L)]
                g = jnp.where(idv >= 0, idv + cbase, ZSLOT)
                idx3[pl.ds(i * L, L)] = g
                idx3[pl.ds(SUBC + i * L, L)] = g + N
                idx3[pl.ds(2 * SUBC + i * L, L)] = g + 2 * N
                return 0
            lax.fori_loop(0, SUBC // L, bld, 0)

            @pl.when(sc_i > 0)
            def _():
                for ch in range(3):
                    pltpu.make_async_copy(
                        cstage.at[pl.ds((1 - cpar) * 3 * SUBC + ch * SUBC,
                                        SUBC)],
                        img_hbm.at[pl.ds(0, SUBC)], sem_im).wait()

            cps = [pltpu.async_copy(
                       colors_hbm.at[idx3.at[pl.ds(ch * SUBC, SUBC)]],
                       cstage.at[pl.ds(cpar * 3 * SUBC + ch * SUBC, SUBC)],
                       sem_cg)
                   for ch in range(3)]
            for cp in cps:
                cp.wait()
            for ch in range(3):
                pltpu.async_copy(
                    cstage.at[pl.ds(cpar * 3 * SUBC + ch * SUBC, SUBC)],
                    img_hbm.at[pl.ds(ibase0 + ch * NREG * R + sbase, SUBC)],
                    sem_im)
            return 0
        lax.fori_loop(0, R // SUBC, sub_body, 0)

        last = (R // SUBC - 1) & 1
        for ch in range(3):
            pltpu.make_async_copy(
                cstage.at[pl.ds(last * 3 * SUBC + ch * SUBC, SUBC)],
                img_hbm.at[pl.ds(0, SUBC)], sem_im).wait()
        cp_d.wait()
        cp_i.wait()


def kernel(points, colors):
    cflat = jnp.pad(colors.reshape(-1), (0, 2 * N + 8))
    depth, img, index = _sc_project(points.reshape(-1), cflat)
    return (depth.reshape(B, H, W),
            img.reshape(B, 3, H, W),
            index.reshape(B, H, W))


# speedup vs baseline: 4.6464x; 4.6464x over previous
import functools

import jax
import jax.numpy as jnp
from jax import lax
from jax.experimental import pallas as pl
from jax.experimental.pallas import tpu as pltpu
from jax.experimental.pallas import tpu_sc as plsc

H, W = 512, 512
HW = H * W
B = 8
N = 131072
NREG = 8
R = HW // NREG
NTASK = B * NREG
CHUNK = 8192
L = 16

_mesh = plsc.VectorSubcoreMesh(core_axis_name="c", subcore_axis_name="s")


def _pix_body(pts, pix):
    x = pts[0]
    y = pts[1]
    u = jnp.minimum((x * jnp.float32(W)).astype(jnp.int32), W - 1)
    v = jnp.minimum((y * jnp.float32(H)).astype(jnp.int32), H - 1)
    pix[0] = (v << 9) | u


_tc_pix = pl.pallas_call(
    _pix_body,
    grid=(B,),
    in_specs=[pl.BlockSpec((4, 1024, 128), lambda b: (b, 0, 0))],
    out_specs=pl.BlockSpec((1, 1024, 128), lambda b: (b, 0, 0)),
    out_shape=jax.ShapeDtypeStruct((B, 1024, 128), jnp.int32),
)


@functools.partial(
    pl.kernel,
    mesh=_mesh,
    out_type=[
        jax.ShapeDtypeStruct((B * NREG * R,), jnp.float32),
        jax.ShapeDtypeStruct((B * 3 * NREG * R,), jnp.float32),
        jax.ShapeDtypeStruct((B * NREG * R,), jnp.int32),
    ],
    scratch_types=[
        pltpu.VMEM((CHUNK,), jnp.int32),
        pltpu.VMEM((CHUNK,), jnp.float32),
        pltpu.VMEM((R,), jnp.float32),
        pltpu.VMEM((R,), jnp.int32),
        pltpu.VMEM((2048,), jnp.int32),
        pltpu.VMEM((CHUNK,), jnp.float32),
        pltpu.VMEM((R,), jnp.float32),
    ],
    compiler_params=pltpu.CompilerParams(needs_layout_passes=False),
)
def _sc_project(points_hbm, pix_hbm, colors_hbm,
                depth_hbm, img_hbm, idx_hbm,
                pixb, zb, depth_v, id_v, tmp_v, cb2, img2):
    wid = lax.axis_index("s") * 2 + lax.axis_index("c")
    lane = lax.iota(jnp.int32, L)
    inf16 = jnp.full((L,), jnp.inf, dtype=jnp.float32)
    zero16 = jnp.zeros((L,), dtype=jnp.float32)
    n16 = jnp.full((L,), N, dtype=jnp.int32)
    nc = N // CHUNK

    for k in range(NTASK // 32):
        t = wid + 32 * k
        b = t >> 3
        q = t & 7
        pbase = b * 4 * N
        cbase = b * 3 * N

        def init_body(i, _):
            depth_v[pl.ds(i * L, L)] = inf16
            id_v[pl.ds(i * L, L)] = n16
            return 0
        lax.fori_loop(0, R // L, init_body, 0)

        def chunk_body(c, _):
            off = c * CHUNK
            pltpu.sync_copy(pix_hbm.at[pl.ds(b * N + off, CHUNK)], pixb)
            pltpu.sync_copy(points_hbm.at[pl.ds(pbase + 2 * N + off, CHUNK)],
                            zb)

            def process(jbase, lex, dirty):
                s = pl.ds(jbase, L)
                pix = pixb[s]
                z = zb[s]
                in_reg = (pix >> 15) == q
                local = pix & (R - 1)
                ids = (off + jbase) + lane
                d0 = plsc.load_gather(depth_v, [local])
                if lex:
                    i0 = plsc.load_gather(id_v, [local])
                    want = in_reg & ((z < d0) | ((z == d0) & (ids < i0)))
                else:
                    want = in_reg & (z < d0)
                slot = local & 2047
                plsc.store_scatter(tmp_v, [slot], lane, mask=want)
                winner = plsc.load_gather(tmp_v, [slot])
                lead = want & (winner == lane)
                plsc.store_scatter(depth_v, [local], z, mask=lead)
                plsc.store_scatter(id_v, [local], ids, mask=lead)
                return dirty | (want ^ lead)

            def make_pass(lex):
                def vec_body(j, dirty):
                    dirty = process(j * 2 * L, lex, dirty)
                    dirty = process(j * 2 * L + L, lex, dirty)
                    return dirty
                return lax.fori_loop(0, CHUNK // (2 * L), vec_body,
                                     jnp.zeros((L,), dtype=jnp.bool_))

            dirty = make_pass(lex=False)
            lax.while_loop(jnp.any, lambda d: make_pass(lex=True), dirty)
            return 0
        lax.fori_loop(0, nc, chunk_body, 0)

        def fin_body(i, _):
            s = pl.ds(i * L, L)
            idv = id_v[s]
            dv = depth_v[s]
            valid = idv < N
            id_v[s] = jnp.where(valid, idv, -1)
            depth_v[s] = jnp.where(valid, dv, 0.0)
            img2[s] = zero16
            return 0
        lax.fori_loop(0, R // L, fin_body, 0)

        obase = (b * NREG + q) * R
        pltpu.sync_copy(depth_v, depth_hbm.at[pl.ds(obase, R)])
        pltpu.sync_copy(id_v, idx_hbm.at[pl.ds(obase, R)])

        def cchunk2(c, _):
            off = c * CHUNK
            pltpu.sync_copy(pix_hbm.at[pl.ds(b * N + off, CHUNK)], pixb)
            pltpu.sync_copy(colors_hbm.at[pl.ds(cbase + off, CHUNK)], zb)
            pltpu.sync_copy(colors_hbm.at[pl.ds(cbase + N + off, CHUNK)],
                            cb2)

            def cprocA(jbase):
                s = pl.ds(jbase, L)
                pix = pixb[s]
                in_reg = (pix >> 15) == q
                local = pix & (R - 1)
                ids = (off + jbase) + lane
                widv = plsc.load_gather(id_v, [local])
                win = in_reg & (widv == ids)
                plsc.store_scatter(depth_v, [local], zb[s], mask=win)
                plsc.store_scatter(img2, [local], cb2[s], mask=win)

            def cvecA(j, _):
                cprocA(j * 2 * L)
                cprocA(j * 2 * L + L)
                return 0
            lax.fori_loop(0, CHUNK // (2 * L), cvecA, 0)
            return 0
        lax.fori_loop(0, nc, cchunk2, 0)

        ibase = (b * 3 * NREG + q) * R
        pltpu.sync_copy(depth_v, img_hbm.at[pl.ds(ibase, R)])
        pltpu.sync_copy(img2, img_hbm.at[pl.ds(ibase + NREG * R, R)])

        def cchunk1(c, _):
            off = c * CHUNK
            pltpu.sync_copy(pix_hbm.at[pl.ds(b * N + off, CHUNK)], pixb)
            pltpu.sync_copy(
                colors_hbm.at[pl.ds(cbase + 2 * N + off, CHUNK)], zb)

            def cprocB(jbase):
                s = pl.ds(jbase, L)
                pix = pixb[s]
                cv = zb[s]
                in_reg = (pix >> 15) == q
                local = pix & (R - 1)
                ids = (off + jbase) + lane
                widv = plsc.load_gather(id_v, [local])
                win = in_reg & (widv == ids)
                plsc.store_scatter(depth_v, [local], cv, mask=win)

            def cvecB(j, _):
                cprocB(j * 2 * L)
                cprocB(j * 2 * L + L)
                return 0
            lax.fori_loop(0, CHUNK // (2 * L), cvecB, 0)
            return 0
        lax.fori_loop(0, nc, cchunk1, 0)

        pltpu.sync_copy(depth_v, img_hbm.at[pl.ds(ibase + 2 * NREG * R, R)])


def kernel(points, colors):
    pix = _tc_pix(points.reshape(B * 4, 1024, 128))
    depth, img, index = _sc_project(points.reshape(-1), pix.reshape(-1),
                                    colors.reshape(-1))
    return (depth.reshape(B, H, W),
            img.reshape(B, 3, H, W),
            index.reshape(B, H, W))
